# Initial kernel scaffold; baseline (speedup 1.0000x reference)
import jax
import jax.numpy as jnp
from jax.experimental import pallas as pl


def kernel(x, edge_index, W1, b1, W2, b2):
    raise NotImplementedError("write your pallas kernel here")


# speedup vs baseline: 3.0794x; 3.0794x over previous
import functools

import jax
import jax.numpy as jnp
from jax.experimental import pallas as pl
from jax.experimental.pallas import tpu as pltpu

N_BLK = 1000


def _tc1_body(x_ref, w1_ref, deg_ref, xws_ref, dinv_ref):
    deg = deg_ref[...]
    dinv = jax.lax.rsqrt(deg)
    xw = jnp.dot(x_ref[...], w1_ref[...], preferred_element_type=jnp.float32)
    xws_ref[...] = xw * dinv
    dinv_ref[...] = dinv


def _tc1(x, w1, deg2d):
    n = x.shape[0]
    grid = (n // N_BLK,)
    return pl.pallas_call(
        _tc1_body,
        grid=grid,
        in_specs=[
            pl.BlockSpec((N_BLK, x.shape[1]), lambda i: (i, 0)),
            pl.BlockSpec((w1.shape[0], w1.shape[1]), lambda i: (0, 0)),
            pl.BlockSpec((N_BLK, 1), lambda i: (i, 0)),
        ],
        out_specs=[
            pl.BlockSpec((N_BLK, w1.shape[1]), lambda i: (i, 0)),
            pl.BlockSpec((N_BLK, 1), lambda i: (i, 0)),
        ],
        out_shape=[
            jax.ShapeDtypeStruct((n, w1.shape[1]), jnp.float32),
            jax.ShapeDtypeStruct((n, 1), jnp.float32),
        ],
    )(x, w1, deg2d)


def _tc2_body(scat_ref, xws_ref, dinv_ref, w2_ref, b1_ref, gs_ref):
    dinv = dinv_ref[...]
    agg = (scat_ref[...] + xws_ref[...]) * dinv + b1_ref[...]
    h = jnp.maximum(agg, 0.0)
    g = jnp.dot(h, w2_ref[...], preferred_element_type=jnp.float32)
    gs_ref[...] = g * dinv


def _tc2(scat, xws, dinv, w2, b1):
    n = scat.shape[0]
    grid = (n // N_BLK,)
    out_w = w2.shape[1]
    b1r = b1.reshape(1, -1)
    return pl.pallas_call(
        _tc2_body,
        grid=grid,
        in_specs=[
            pl.BlockSpec((N_BLK, scat.shape[1]), lambda i: (i, 0)),
            pl.BlockSpec((N_BLK, xws.shape[1]), lambda i: (i, 0)),
            pl.BlockSpec((N_BLK, 1), lambda i: (i, 0)),
            pl.BlockSpec((w2.shape[0], out_w), lambda i: (0, 0)),
            pl.BlockSpec((1, b1r.shape[1]), lambda i: (0, 0)),
        ],
        out_specs=pl.BlockSpec((N_BLK, out_w), lambda i: (i, 0)),
        out_shape=jax.ShapeDtypeStruct((n, out_w), jnp.float32),
    )(scat, xws, dinv, w2, b1r)


def _tc3_body(scat2_ref, gs_ref, dinv_ref, b2_ref, out_ref):
    dinv = dinv_ref[...]
    logits = (scat2_ref[...] + gs_ref[...]) * dinv + b2_ref[...]
    m = jnp.max(logits, axis=1, keepdims=True)
    e = jnp.exp(logits - m)
    out_ref[...] = e / jnp.sum(e, axis=1, keepdims=True)


def _tc3(scat2, gs, dinv, b2):
    n = scat2.shape[0]
    grid = (n // N_BLK,)
    w = scat2.shape[1]
    b2r = b2.reshape(1, -1)
    return pl.pallas_call(
        _tc3_body,
        grid=grid,
        in_specs=[
            pl.BlockSpec((N_BLK, w), lambda i: (i, 0)),
            pl.BlockSpec((N_BLK, w), lambda i: (i, 0)),
            pl.BlockSpec((N_BLK, 1), lambda i: (i, 0)),
            pl.BlockSpec((1, b2r.shape[1]), lambda i: (0, 0)),
        ],
        out_specs=pl.BlockSpec((N_BLK, w), lambda i: (i, 0)),
        out_shape=jax.ShapeDtypeStruct((n, w), jnp.float32),
    )(scat2, gs, dinv, b2r)


def kernel(x, edge_index, W1, b1, W2, b2):
    n = x.shape[0]
    src = edge_index[0]
    dst = edge_index[1]
    deg = jnp.ones((n,), jnp.float32).at[dst].add(1.0)
    deg2d = deg.reshape(n, 1)
    xws, dinv = _tc1(x, W1, deg2d)
    scat = jnp.zeros((n, xws.shape[1]), jnp.float32).at[dst].add(xws[src])
    gs = _tc2(scat, xws, dinv, W2, b1)
    scat2 = jnp.zeros((n, gs.shape[1]), jnp.float32).at[dst].add(gs[src])
    return _tc3(scat2, gs, dinv, b2)


# speedup vs baseline: 13.6847x; 4.4439x over previous
import functools

import jax
import jax.numpy as jnp
from jax import lax
from jax.experimental import pallas as pl
from jax.experimental.pallas import tpu as pltpu
from jax.experimental.pallas import tpu_sc as plsc

NC = 2
NS = 16
NW = NC * NS
K = 128

N_BLK = 1000


def _wid(c, s):
    return c * NS + s


def _make_agg(n_pad, width, ew, rpt):
    n_chunks = ew // K
    mesh = plsc.VectorSubcoreMesh(core_axis_name="c", subcore_axis_name="s")

    @functools.partial(
        pl.kernel,
        out_type=jax.ShapeDtypeStruct((NC, n_pad, width), jnp.float32),
        mesh=mesh,
        compiler_params=pltpu.CompilerParams(use_tc_tiling_on_sc=False),
        scratch_types=[
            pltpu.VMEM((K,), jnp.int32),
            pltpu.VMEM((K,), jnp.int32),
            pltpu.VMEM((K, width), jnp.float32),
            pltpu.VMEM_SHARED((n_pad, width), jnp.float32),
            pltpu.SemaphoreType.DMA,
        ],
    )
    def agg(table_hbm, srcp_hbm, dstp_hbm, zeros_hbm, out_hbm,
            src_v, dst_v, rows_v, acc_sh, sem):
        c = lax.axis_index("c")
        s = lax.axis_index("s")
        pltpu.sync_copy(zeros_hbm, acc_sh.at[pl.ds(s * rpt, rpt)])
        plsc.subcore_barrier()
        base = _wid(c, s) * ew

        def chunk(i, carry):
            off = base + i * K
            pltpu.sync_copy(srcp_hbm.at[pl.ds(off, K)], src_v)
            pltpu.sync_copy(dstp_hbm.at[pl.ds(off, K)], dst_v)
            pltpu.async_copy(table_hbm.at[src_v], rows_v, sem).wait()
            pltpu.sync_copy(rows_v, acc_sh.at[dst_v], add=True)
            return carry

        lax.fori_loop(0, n_chunks, chunk, 0)
        plsc.subcore_barrier()
        pltpu.sync_copy(acc_sh.at[pl.ds(s * rpt, rpt)],
                        out_hbm.at[c, pl.ds(s * rpt, rpt)])

    return agg


def _make_deg(n_pad, ew, rpt):
    n_chunks = ew // K
    mesh = plsc.VectorSubcoreMesh(core_axis_name="c", subcore_axis_name="s")

    @functools.partial(
        pl.kernel,
        out_type=jax.ShapeDtypeStruct((NC, n_pad), jnp.float32),
        mesh=mesh,
        scratch_types=[
            pltpu.VMEM((K,), jnp.int32),
            pltpu.VMEM((K,), jnp.float32),
            pltpu.VMEM_SHARED((n_pad,), jnp.float32),
        ],
    )
    def deg(dstp_hbm, zeros_hbm, out_hbm, dst_v, ones_v, acc_sh):
        c = lax.axis_index("c")
        s = lax.axis_index("s")
        for j in range(K // 16):
            ones_v[pl.ds(16 * j, 16)] = jnp.ones((16,), jnp.float32)
        pltpu.sync_copy(zeros_hbm, acc_sh.at[pl.ds(s * rpt, rpt)])
        plsc.subcore_barrier()
        base = _wid(c, s) * ew

        def chunk(i, carry):
            off = base + i * K
            pltpu.sync_copy(dstp_hbm.at[pl.ds(off, K)], dst_v)
            pltpu.sync_copy(ones_v, acc_sh.at[dst_v], add=True)
            return carry

        lax.fori_loop(0, n_chunks, chunk, 0)
        plsc.subcore_barrier()
        pltpu.sync_copy(acc_sh.at[pl.ds(s * rpt, rpt)],
                        out_hbm.at[c, pl.ds(s * rpt, rpt)])

    return deg


def _tc1_body(x_ref, w1_ref, deg_ref, xws_ref):
    dinv = jax.lax.rsqrt(deg_ref[...])
    xw = jnp.dot(x_ref[...], w1_ref[...], preferred_element_type=jnp.float32)
    xws_ref[...] = xw * dinv


def _tc1(x, w1, deg2d):
    n, d = x.shape
    return pl.pallas_call(
        _tc1_body,
        grid=(n // N_BLK,),
        in_specs=[
            pl.BlockSpec((N_BLK, d), lambda i: (i, 0)),
            pl.BlockSpec((d, w1.shape[1]), lambda i: (0, 0)),
            pl.BlockSpec((N_BLK, 1), lambda i: (i, 0)),
        ],
        out_specs=pl.BlockSpec((N_BLK, w1.shape[1]), lambda i: (i, 0)),
        out_shape=jax.ShapeDtypeStruct((n, w1.shape[1]), jnp.float32),
    )(x, w1, deg2d)


def _tc2_body(p_ref, xws_ref, deg_ref, w2_ref, b1_ref, gs_ref):
    dinv = jax.lax.rsqrt(deg_ref[...])
    scat = p_ref[0] + p_ref[1]
    agg = (scat + xws_ref[...]) * dinv + b1_ref[...]
    h = jnp.maximum(agg, 0.0)
    g = jnp.dot(h, w2_ref[...], preferred_element_type=jnp.float32)
    gs_ref[...] = g * dinv


def _tc2(partials, xws, deg2d, w2p, b1):
    n, hdim = xws.shape
    wout = w2p.shape[1]
    b1r = b1.reshape(1, -1)
    return pl.pallas_call(
        _tc2_body,
        grid=(n // N_BLK,),
        in_specs=[
            pl.BlockSpec((NC, N_BLK, hdim), lambda i: (0, i, 0)),
            pl.BlockSpec((N_BLK, hdim), lambda i: (i, 0)),
            pl.BlockSpec((N_BLK, 1), lambda i: (i, 0)),
            pl.BlockSpec((hdim, wout), lambda i: (0, 0)),
            pl.BlockSpec((1, hdim), lambda i: (0, 0)),
        ],
        out_specs=pl.BlockSpec((N_BLK, wout), lambda i: (i, 0)),
        out_shape=jax.ShapeDtypeStruct((n, wout), jnp.float32),
    )(partials, xws, deg2d, w2p, b1r)


def _tc3_body(p_ref, gs_ref, deg_ref, b2_ref, out_ref):
    dinv = jax.lax.rsqrt(deg_ref[...])
    scat = p_ref[0] + p_ref[1]
    logits = (scat + gs_ref[...]) * dinv
    l0 = logits[:, 0:1] + b2_ref[0, 0]
    l1 = logits[:, 1:2] + b2_ref[0, 1]
    m = jnp.maximum(l0, l1)
    e0 = jnp.exp(l0 - m)
    e1 = jnp.exp(l1 - m)
    denom = e0 + e1
    out_ref[...] = jnp.concatenate([e0 / denom, e1 / denom], axis=1)


def _tc3(partials2, gs, deg2d, b2):
    n, wpad = gs.shape
    b2r = b2.reshape(1, -1)
    return pl.pallas_call(
        _tc3_body,
        grid=(n // N_BLK,),
        in_specs=[
            pl.BlockSpec((NC, N_BLK, wpad), lambda i: (0, i, 0)),
            pl.BlockSpec((N_BLK, wpad), lambda i: (i, 0)),
            pl.BlockSpec((N_BLK, 1), lambda i: (i, 0)),
            pl.BlockSpec((1, b2r.shape[1]), lambda i: (0, 0)),
        ],
        out_specs=pl.BlockSpec((N_BLK, 2), lambda i: (i, 0)),
        out_shape=jax.ShapeDtypeStruct((n, 2), jnp.float32),
    )(partials2, gs, deg2d, b2r)


def kernel(x, edge_index, W1, b1, W2, b2):
    n, d = x.shape
    e = edge_index.shape[1]
    n_pad = 10240
    rpt = n_pad // NS
    ew = -(-e // (NW * K)) * K
    e_pad = ew * NW

    src = edge_index[0]
    dst = edge_index[1]
    pad = e_pad - e
    srcp = jnp.concatenate([src, jnp.zeros((pad,), src.dtype)])
    dstp = jnp.concatenate([dst, jnp.full((pad,), n, dst.dtype)])

    zeros_w = jnp.zeros((rpt, d), jnp.float32)
    zeros_16 = jnp.zeros((rpt, 16), jnp.float32)
    zeros_1 = jnp.zeros((rpt,), jnp.float32)

    degp = _make_deg(n_pad, ew, rpt)(dstp, zeros_1)
    deg2d = (degp[0, :n] + degp[1, :n] + 1.0).reshape(n, 1)

    xws = _tc1(x, W1, deg2d)
    partials = _make_agg(n_pad, d, ew, rpt)(xws, srcp, dstp, zeros_w)

    w2p = jnp.zeros((d, 16), W2.dtype).at[:, : W2.shape[1]].set(W2)
    gs = _tc2(partials, xws, deg2d, w2p, b1)
    partials2 = _make_agg(n_pad, 16, ew, rpt)(gs, srcp, dstp, zeros_16)
    return _tc3(partials2, gs, deg2d, b2)


# speedup vs baseline: 18.9702x; 1.3862x over previous
import functools

import jax
import jax.numpy as jnp
from jax import lax
from jax.experimental import pallas as pl
from jax.experimental.pallas import tpu as pltpu
from jax.experimental.pallas import tpu_sc as plsc

NC = 2
NS = 16
NW = NC * NS
K = 128
IB = 40

N_BLK = 1000

_MESH = dict(core_axis_name="c", subcore_axis_name="s")


def _make_agg_wsplit(n_pad, half, rpt, g, cpt):
    mesh = plsc.VectorSubcoreMesh(**_MESH)

    @functools.partial(
        pl.kernel,
        out_type=jax.ShapeDtypeStruct((NC, n_pad, half), jnp.float32),
        mesh=mesh,
        compiler_params=pltpu.CompilerParams(use_tc_tiling_on_sc=False),
        scratch_types=[
            pltpu.VMEM((IB, K), jnp.int32),
            pltpu.VMEM((IB, K), jnp.int32),
            pltpu.VMEM((g, K, half), jnp.float32),
            pltpu.VMEM_SHARED((n_pad, half), jnp.float32),
            pltpu.SemaphoreType.DMA,
            pltpu.SemaphoreType.DMA,
        ],
    )
    def agg(table_hbm, src2d_hbm, dst2d_hbm, zeros_hbm, out_hbm,
            sblk, dblk, rows_v, acc_sh, gsem, ssem):
        c = lax.axis_index("c")
        s = lax.axis_index("s")
        pltpu.sync_copy(zeros_hbm, acc_sh.at[pl.ds(s * rpt, rpt)])
        plsc.subcore_barrier()
        wrow = s * cpt

        def iblock(ib, carry):
            row0 = wrow + ib * IB
            pltpu.sync_copy(src2d_hbm.at[pl.ds(row0, IB)], sblk)
            pltpu.sync_copy(dst2d_hbm.at[pl.ds(row0, IB)], dblk)

            def group(gi, carry2):
                j0 = gi * g
                gds = [
                    pltpu.async_copy(table_hbm.at[c].at[sblk.at[j0 + u]],
                                     rows_v.at[u], gsem)
                    for u in range(g)
                ]
                for d in gds:
                    d.wait()
                sds = [
                    pltpu.async_copy(rows_v.at[u],
                                     acc_sh.at[dblk.at[j0 + u]],
                                     ssem, add=True)
                    for u in range(g)
                ]
                for d in sds:
                    d.wait()
                return carry2

            lax.fori_loop(0, IB // g, group, 0)
            return carry

        lax.fori_loop(0, cpt // IB, iblock, 0)
        plsc.subcore_barrier()
        pltpu.sync_copy(acc_sh.at[pl.ds(s * rpt, rpt)],
                        out_hbm.at[c, pl.ds(s * rpt, rpt)])

    return agg


def _make_agg_esplit(n_pad, width, rpt, g, cpw):
    mesh = plsc.VectorSubcoreMesh(**_MESH)

    @functools.partial(
        pl.kernel,
        out_type=jax.ShapeDtypeStruct((NC, n_pad, width), jnp.float32),
        mesh=mesh,
        compiler_params=pltpu.CompilerParams(use_tc_tiling_on_sc=False),
        scratch_types=[
            pltpu.VMEM((IB, K), jnp.int32),
            pltpu.VMEM((IB, K), jnp.int32),
            pltpu.VMEM((g, K, width), jnp.float32),
            pltpu.VMEM_SHARED((n_pad, width), jnp.float32),
            pltpu.SemaphoreType.DMA,
            pltpu.SemaphoreType.DMA,
        ],
    )
    def agg(table_hbm, src2d_hbm, dst2d_hbm, zeros_hbm, out_hbm,
            sblk, dblk, rows_v, acc_sh, gsem, ssem):
        c = lax.axis_index("c")
        s = lax.axis_index("s")
        pltpu.sync_copy(zeros_hbm, acc_sh.at[pl.ds(s * rpt, rpt)])
        plsc.subcore_barrier()
        wrow = (c * NS + s) * cpw

        def iblock(ib, carry):
            row0 = wrow + ib * IB
            pltpu.sync_copy(src2d_hbm.at[pl.ds(row0, IB)], sblk)
            pltpu.sync_copy(dst2d_hbm.at[pl.ds(row0, IB)], dblk)

            def group(gi, carry2):
                j0 = gi * g
                gds = [
                    pltpu.async_copy(table_hbm.at[sblk.at[j0 + u]],
                                     rows_v.at[u], gsem)
                    for u in range(g)
                ]
                for d in gds:
                    d.wait()
                sds = [
                    pltpu.async_copy(rows_v.at[u],
                                     acc_sh.at[dblk.at[j0 + u]],
                                     ssem, add=True)
                    for u in range(g)
                ]
                for d in sds:
                    d.wait()
                return carry2

            lax.fori_loop(0, IB // g, group, 0)
            return carry

        lax.fori_loop(0, cpw // IB, iblock, 0)
        plsc.subcore_barrier()
        pltpu.sync_copy(acc_sh.at[pl.ds(s * rpt, rpt)],
                        out_hbm.at[c, pl.ds(s * rpt, rpt)])

    return agg


def _make_deg(n_pad, rpt, cpw):
    mesh = plsc.VectorSubcoreMesh(**_MESH)

    @functools.partial(
        pl.kernel,
        out_type=jax.ShapeDtypeStruct((NC, n_pad), jnp.float32),
        mesh=mesh,
        scratch_types=[
            pltpu.VMEM((IB, K), jnp.int32),
            pltpu.VMEM((K,), jnp.float32),
            pltpu.VMEM_SHARED((n_pad,), jnp.float32),
            pltpu.SemaphoreType.DMA,
        ],
    )
    def deg(dst2d_hbm, zeros_hbm, out_hbm, dblk, ones_v, acc_sh, ssem):
        c = lax.axis_index("c")
        s = lax.axis_index("s")
        for j in range(K // 16):
            ones_v[pl.ds(16 * j, 16)] = jnp.ones((16,), jnp.float32)
        pltpu.sync_copy(zeros_hbm, acc_sh.at[pl.ds(s * rpt, rpt)])
        plsc.subcore_barrier()
        wrow = (c * NS + s) * cpw

        def iblock(ib, carry):
            row0 = wrow + ib * IB
            pltpu.sync_copy(dst2d_hbm.at[pl.ds(row0, IB)], dblk)

            def group(gi, carry2):
                j0 = gi * 10
                sds = [
                    pltpu.async_copy(ones_v, acc_sh.at[dblk.at[j0 + u]],
                                     ssem, add=True)
                    for u in range(10)
                ]
                for d in sds:
                    d.wait()
                return carry2

            lax.fori_loop(0, IB // 10, group, 0)
            return carry

        lax.fori_loop(0, cpw // IB, iblock, 0)
        plsc.subcore_barrier()
        pltpu.sync_copy(acc_sh.at[pl.ds(s * rpt, rpt)],
                        out_hbm.at[c, pl.ds(s * rpt, rpt)])

    return deg


def _tc1_body(x_ref, w1_ref, deg_ref, xws_ref):
    dinv = jax.lax.rsqrt(deg_ref[...])
    xw = jnp.dot(x_ref[...], w1_ref[...], preferred_element_type=jnp.float32)
    xws = xw * dinv
    xws_ref[0] = xws[:, :64]
    xws_ref[1] = xws[:, 64:]


def _tc1(x, w1, deg2d):
    n, d = x.shape
    return pl.pallas_call(
        _tc1_body,
        grid=(n // N_BLK,),
        in_specs=[
            pl.BlockSpec((N_BLK, d), lambda i: (i, 0)),
            pl.BlockSpec((d, w1.shape[1]), lambda i: (0, 0)),
            pl.BlockSpec((N_BLK, 1), lambda i: (i, 0)),
        ],
        out_specs=pl.BlockSpec((NC, N_BLK, d // 2), lambda i: (0, i, 0)),
        out_shape=jax.ShapeDtypeStruct((NC, n, d // 2), jnp.float32),
    )(x, w1, deg2d)


def _tc2_body(p_ref, xws_ref, deg_ref, w2_ref, b1_ref, gs_ref):
    dinv = jax.lax.rsqrt(deg_ref[...])
    scat = jnp.concatenate([p_ref[0], p_ref[1]], axis=1)
    xws = jnp.concatenate([xws_ref[0], xws_ref[1]], axis=1)
    agg = (scat + xws) * dinv + b1_ref[...]
    h = jnp.maximum(agg, 0.0)
    g = jnp.dot(h, w2_ref[...], preferred_element_type=jnp.float32)
    gs_ref[...] = g * dinv


def _tc2(partials, xws2, deg2d, w2p, b1):
    nc, n, half = xws2.shape
    hdim = nc * half
    wout = w2p.shape[1]
    b1r = b1.reshape(1, -1)
    return pl.pallas_call(
        _tc2_body,
        grid=(n // N_BLK,),
        in_specs=[
            pl.BlockSpec((NC, N_BLK, half), lambda i: (0, i, 0)),
            pl.BlockSpec((NC, N_BLK, half), lambda i: (0, i, 0)),
            pl.BlockSpec((N_BLK, 1), lambda i: (i, 0)),
            pl.BlockSpec((hdim, wout), lambda i: (0, 0)),
            pl.BlockSpec((1, hdim), lambda i: (0, 0)),
        ],
        out_specs=pl.BlockSpec((N_BLK, wout), lambda i: (i, 0)),
        out_shape=jax.ShapeDtypeStruct((n, wout), jnp.float32),
    )(partials, xws2, deg2d, w2p, b1r)


def _tc3_body(p_ref, gs_ref, deg_ref, b2_ref, out_ref):
    dinv = jax.lax.rsqrt(deg_ref[...])
    scat = p_ref[0] + p_ref[1]
    logits = (scat + gs_ref[...]) * dinv
    l0 = logits[:, 0:1] + b2_ref[0, 0]
    l1 = logits[:, 1:2] + b2_ref[0, 1]
    m = jnp.maximum(l0, l1)
    e0 = jnp.exp(l0 - m)
    e1 = jnp.exp(l1 - m)
    denom = e0 + e1
    out_ref[...] = jnp.concatenate([e0 / denom, e1 / denom], axis=1)


def _tc3(partials2, gs, deg2d, b2):
    n, wpad = gs.shape
    b2r = b2.reshape(1, -1)
    return pl.pallas_call(
        _tc3_body,
        grid=(n // N_BLK,),
        in_specs=[
            pl.BlockSpec((NC, N_BLK, wpad), lambda i: (0, i, 0)),
            pl.BlockSpec((N_BLK, wpad), lambda i: (i, 0)),
            pl.BlockSpec((N_BLK, 1), lambda i: (i, 0)),
            pl.BlockSpec((1, b2r.shape[1]), lambda i: (0, 0)),
        ],
        out_specs=pl.BlockSpec((N_BLK, 2), lambda i: (i, 0)),
        out_shape=jax.ShapeDtypeStruct((n, 2), jnp.float32),
    )(partials2, gs, deg2d, b2r)


def kernel(x, edge_index, W1, b1, W2, b2):
    n, d = x.shape
    e = edge_index.shape[1]
    n_pad = 10240
    rpt = n_pad // NS
    cpw = 80
    cpt = cpw * NC
    e_pad = NW * cpw * K
    pad = e_pad - e

    src = edge_index[0]
    dst = edge_index[1]
    srcp = jnp.concatenate([src, jnp.zeros((pad,), src.dtype)])
    dstp = jnp.concatenate([dst, jnp.full((pad,), n, dst.dtype)])
    src2d = srcp.reshape(-1, K)
    dst2d = dstp.reshape(-1, K)

    zeros_h = jnp.zeros((rpt, d // 2), jnp.float32)
    zeros_16 = jnp.zeros((rpt, 16), jnp.float32)
    zeros_1 = jnp.zeros((rpt,), jnp.float32)

    degp = _make_deg(n_pad, rpt, cpw)(dst2d, zeros_1)
    deg2d = (degp[0, :n] + degp[1, :n] + 1.0).reshape(n, 1)

    xws2 = _tc1(x, W1, deg2d)
    halves = _make_agg_wsplit(n_pad, d // 2, rpt, 8, cpt)(
        xws2, src2d, dst2d, zeros_h)

    w2p = jnp.zeros((d, 16), W2.dtype).at[:, : W2.shape[1]].set(W2)
    gs = _tc2(halves, xws2, deg2d, w2p, b1)
    partials2 = _make_agg_esplit(n_pad, 16, rpt, 8, cpw)(
        gs, src2d, dst2d, zeros_16)
    return _tc3(partials2, gs, deg2d, b2)
